# Initial kernel scaffold; baseline (speedup 1.0000x reference)
import jax
import jax.numpy as jnp
from jax.experimental import pallas as pl


def kernel(node_feats, W_emb, W_mlp1, W_mlp2, W_mlp3, species_embed, W_pos, n_node, target_species):
    raise NotImplementedError("write your pallas kernel here")


# speedup vs baseline: 1.3195x; 1.3195x over previous
import math

import jax
import jax.numpy as jnp
import numpy as np
from jax.experimental import pallas as pl

NUM_GRAPHS = 64
NODES_PER_GRAPH = 32
NUM_NODES = NUM_GRAPHS * NODES_PER_GRAPH
D_IN = 128
NUM_SPECIES = 90
LMAX = 4
N_COEFFS = (LMAX + 1) ** 2
N_RADII = 64
RES_BETA = 40
RES_ALPHA = 79


def _legendre_table(lmax, x):
    n = x.shape[0]
    P = np.zeros((lmax + 1, lmax + 1, n))
    P[0, 0] = 1.0
    somx2 = np.sqrt(np.maximum(1.0 - x * x, 0.0))
    for m in range(1, lmax + 1):
        P[m, m] = -(2 * m - 1) * somx2 * P[m - 1, m - 1]
    for m in range(lmax):
        P[m + 1, m] = x * (2 * m + 1) * P[m, m]
    for m in range(lmax + 1):
        for l in range(m + 2, lmax + 1):
            P[l, m] = ((2 * l - 1) * x * P[l - 1, m] - (l + m - 1) * P[l - 2, m]) / (l - m)
    return P


def _sph_factors(lmax, res_beta, res_alpha):
    x, _ = np.polynomial.legendre.leggauss(res_beta)
    alpha = np.linspace(0.0, 2.0 * np.pi, res_alpha, endpoint=False)
    P = _legendre_table(lmax, x)
    n_c = (lmax + 1) ** 2
    PP = np.zeros((res_beta, n_c))
    TR = np.zeros((n_c, res_alpha))
    for l in range(lmax + 1):
        for m in range(-l, l + 1):
            c = l * l + l + m
            am = abs(m)
            N = math.sqrt((2 * l + 1) / (4.0 * math.pi)
                          * math.factorial(l - am) / math.factorial(l + am))
            if m == 0:
                PP[:, c] = N * P[l, 0]
                TR[c] = 1.0
            elif m > 0:
                PP[:, c] = math.sqrt(2.0) * N * P[l, m]
                TR[c] = np.cos(m * alpha)
            else:
                PP[:, c] = math.sqrt(2.0) * N * P[l, am]
                TR[c] = np.sin(am * alpha)
    return PP.astype(np.float32), TR.astype(np.float32)


_PP, _TRIG = _sph_factors(LMAX, RES_BETA, RES_ALPHA)
_R40 = np.kron(np.eye(N_RADII, dtype=np.float32), np.ones((RES_BETA, 1), np.float32))
_PP_TILE = np.tile(_PP, (N_RADII, 1))


def _silu(x):
    return x / (1.0 + jnp.exp(-x))


def _head_body(nf_ref, wemb_ref, w1_ref, w2_ref, w3_ref, semb_ref, wpos_ref,
               fidx_ref, ts_ref, logits_ref, probs_ref, coeffs_ref):
    ne = jnp.dot(nf_ref[...], wemb_ref[...], preferred_element_type=jnp.float32)
    h = _silu(jnp.dot(ne, w1_ref[...], preferred_element_type=jnp.float32))
    h = _silu(jnp.dot(h, w2_ref[...], preferred_element_type=jnp.float32))
    logits = jnp.dot(h, w3_ref[...], preferred_element_type=jnp.float32)
    logits_ref[...] = logits
    x3 = logits.reshape(NUM_GRAPHS, NODES_PER_GRAPH, NUM_SPECIES + 1)
    m = jnp.max(jnp.max(x3, axis=2, keepdims=True), axis=1, keepdims=True)
    e = jnp.exp(x3 - m)
    norm = jnp.sum(jnp.sum(e, axis=2, keepdims=True), axis=1, keepdims=True)
    probs_ref[...] = (e / norm).reshape(NUM_NODES, NUM_SPECIES + 1)
    oh_f = (jax.lax.broadcasted_iota(jnp.int32, (NUM_GRAPHS, NUM_NODES), 1)
            == fidx_ref[...]).astype(jnp.float32)
    focus = jnp.dot(oh_f, ne, preferred_element_type=jnp.float32,
                    precision=jax.lax.Precision.HIGHEST)
    oh_s = (jax.lax.broadcasted_iota(jnp.int32, (NUM_GRAPHS, NUM_SPECIES), 1)
            == ts_ref[...]).astype(jnp.float32)
    tse = jnp.dot(oh_s, semb_ref[...], preferred_element_type=jnp.float32,
                  precision=jax.lax.Precision.HIGHEST)
    coeffs_ref[...] = jnp.dot(tse * focus, wpos_ref[...],
                              preferred_element_type=jnp.float32)


def _pos_body(coef_ref, r40_ref, ppt_ref, trig_ref, logit_ref, prob_ref):
    cg = coef_ref[0]
    crep = jnp.dot(r40_ref[...], cg, preferred_element_type=jnp.float32, precision=jax.lax.Precision.HIGHEST)
    crep = crep.astype(jnp.bfloat16).astype(jnp.float32)
    e = crep * ppt_ref[...]
    lg = jnp.dot(e, trig_ref[...], preferred_element_type=jnp.float32, precision=jax.lax.Precision.HIGHEST)
    logit_ref[0] = lg
    m = jnp.max(lg)
    prob_ref[0] = jnp.exp(lg - m)


def kernel(node_feats, W_emb, W_mlp1, W_mlp2, W_mlp3, species_embed, W_pos,
           n_node, target_species):
    fidx = jnp.concatenate(
        [jnp.zeros((1,), jnp.int32), jnp.cumsum(n_node)[:-1].astype(jnp.int32)])
    f32 = jnp.float32
    species_logits, species_probs, coeffs = pl.pallas_call(
        _head_body,
        out_shape=[
            jax.ShapeDtypeStruct((NUM_NODES, NUM_SPECIES + 1), f32),
            jax.ShapeDtypeStruct((NUM_NODES, NUM_SPECIES + 1), f32),
            jax.ShapeDtypeStruct((NUM_GRAPHS, N_RADII * N_COEFFS), f32),
        ],
    )(node_feats, W_emb, W_mlp1, W_mlp2, W_mlp3, species_embed, W_pos,
      fidx.reshape(NUM_GRAPHS, 1),
      target_species.astype(jnp.int32).reshape(NUM_GRAPHS, 1))

    position_coeffs = coeffs.reshape(NUM_GRAPHS, N_RADII, N_COEFFS)
    rb = N_RADII * RES_BETA
    lg_flat, pb_flat = pl.pallas_call(
        _pos_body,
        grid=(NUM_GRAPHS,),
        in_specs=[
            pl.BlockSpec((1, N_RADII, N_COEFFS), lambda g: (g, 0, 0)),
            pl.BlockSpec((rb, N_RADII), lambda g: (0, 0)),
            pl.BlockSpec((rb, N_COEFFS), lambda g: (0, 0)),
            pl.BlockSpec((N_COEFFS, RES_ALPHA), lambda g: (0, 0)),
        ],
        out_specs=[
            pl.BlockSpec((1, rb, RES_ALPHA), lambda g: (g, 0, 0)),
            pl.BlockSpec((1, rb, RES_ALPHA), lambda g: (g, 0, 0)),
        ],
        out_shape=[
            jax.ShapeDtypeStruct((NUM_GRAPHS, rb, RES_ALPHA), f32),
            jax.ShapeDtypeStruct((NUM_GRAPHS, rb, RES_ALPHA), f32),
        ],
    )(position_coeffs, jnp.asarray(_R40), jnp.asarray(_PP_TILE),
      jnp.asarray(_TRIG))

    position_logits = lg_flat.reshape(NUM_GRAPHS, N_RADII, RES_BETA, RES_ALPHA)
    position_probs = pb_flat.reshape(NUM_GRAPHS, N_RADII, RES_BETA, RES_ALPHA)
    return (species_logits, species_probs, position_coeffs, position_logits,
            position_probs, fidx)


# speedup vs baseline: 3.8052x; 2.8839x over previous
import math

import jax
import jax.numpy as jnp
import numpy as np
from jax.experimental import pallas as pl

NUM_GRAPHS = 64
NODES_PER_GRAPH = 32
NUM_NODES = NUM_GRAPHS * NODES_PER_GRAPH
D_IN = 128
NUM_SPECIES = 90
LMAX = 4
N_COEFFS = (LMAX + 1) ** 2
N_RADII = 64
RES_BETA = 40
RES_ALPHA = 79


def _legendre_table(lmax, x):
    n = x.shape[0]
    P = np.zeros((lmax + 1, lmax + 1, n))
    P[0, 0] = 1.0
    somx2 = np.sqrt(np.maximum(1.0 - x * x, 0.0))
    for m in range(1, lmax + 1):
        P[m, m] = -(2 * m - 1) * somx2 * P[m - 1, m - 1]
    for m in range(lmax):
        P[m + 1, m] = x * (2 * m + 1) * P[m, m]
    for m in range(lmax + 1):
        for l in range(m + 2, lmax + 1):
            P[l, m] = ((2 * l - 1) * x * P[l - 1, m] - (l + m - 1) * P[l - 2, m]) / (l - m)
    return P


def _sph_factors(lmax, res_beta, res_alpha):
    x, _ = np.polynomial.legendre.leggauss(res_beta)
    alpha = np.linspace(0.0, 2.0 * np.pi, res_alpha, endpoint=False)
    P = _legendre_table(lmax, x)
    n_c = (lmax + 1) ** 2
    PP = np.zeros((res_beta, n_c))
    TR = np.zeros((n_c, res_alpha))
    for l in range(lmax + 1):
        for m in range(-l, l + 1):
            c = l * l + l + m
            am = abs(m)
            N = math.sqrt((2 * l + 1) / (4.0 * math.pi)
                          * math.factorial(l - am) / math.factorial(l + am))
            if m == 0:
                PP[:, c] = N * P[l, 0]
                TR[c] = 1.0
            elif m > 0:
                PP[:, c] = math.sqrt(2.0) * N * P[l, m]
                TR[c] = np.cos(m * alpha)
            else:
                PP[:, c] = math.sqrt(2.0) * N * P[l, am]
                TR[c] = np.sin(am * alpha)
    return PP.astype(np.float32), TR.astype(np.float32)


_PP, _TRIG = _sph_factors(LMAX, RES_BETA, RES_ALPHA)
import ml_dtypes as _mld
_TRIG_HI = _TRIG.astype(_mld.bfloat16).astype(np.float32)
_TRIG_LO = (_TRIG - _TRIG_HI).astype(_mld.bfloat16).astype(np.float32)
_R40 = np.kron(np.eye(N_RADII, dtype=np.float32), np.ones((RES_BETA, 1), np.float32))
_PP_TILE = np.tile(_PP, (N_RADII, 1))


def _silu(x):
    return x / (1.0 + jnp.exp(-x))


def _head_body(nf_ref, wemb_ref, w1_ref, w2_ref, w3_ref, semb_ref, wpos_ref,
               fidx_ref, ts_ref, logits_ref, probs_ref, coeffs_ref):
    ne = jnp.dot(nf_ref[...], wemb_ref[...], preferred_element_type=jnp.float32)
    h = _silu(jnp.dot(ne, w1_ref[...], preferred_element_type=jnp.float32))
    h = _silu(jnp.dot(h, w2_ref[...], preferred_element_type=jnp.float32))
    logits = jnp.dot(h, w3_ref[...], preferred_element_type=jnp.float32)
    logits_ref[...] = logits
    x3 = logits.reshape(NUM_GRAPHS, NODES_PER_GRAPH, NUM_SPECIES + 1)
    m = jnp.max(jnp.max(x3, axis=2, keepdims=True), axis=1, keepdims=True)
    e = jnp.exp(x3 - m)
    norm = jnp.sum(jnp.sum(e, axis=2, keepdims=True), axis=1, keepdims=True)
    probs_ref[...] = (e / norm).reshape(NUM_NODES, NUM_SPECIES + 1)
    oh_f = (jax.lax.broadcasted_iota(jnp.int32, (NUM_GRAPHS, NUM_NODES), 1)
            == fidx_ref[...]).astype(jnp.float32)
    focus = jnp.dot(oh_f, ne, preferred_element_type=jnp.float32,
                    precision=jax.lax.Precision.HIGHEST)
    oh_s = (jax.lax.broadcasted_iota(jnp.int32, (NUM_GRAPHS, NUM_SPECIES), 1)
            == ts_ref[...]).astype(jnp.float32)
    tse = jnp.dot(oh_s, semb_ref[...], preferred_element_type=jnp.float32,
                  precision=jax.lax.Precision.HIGHEST)
    coeffs_ref[...] = jnp.dot(tse * focus, wpos_ref[...],
                              preferred_element_type=jnp.float32)


def _pos_body(coef_ref, r40_ref, ppt_ref, trig_hi_ref, trig_lo_ref,
              logit_ref, prob_ref):
    cg = coef_ref[0]
    crep = jnp.dot(r40_ref[...], cg, preferred_element_type=jnp.float32)
    e = crep * ppt_ref[...]
    lg = (jnp.dot(e, trig_hi_ref[...], preferred_element_type=jnp.float32)
          + jnp.dot(e, trig_lo_ref[...], preferred_element_type=jnp.float32))
    logit_ref[0] = lg
    m = jnp.max(lg)
    prob_ref[0] = jnp.exp(lg - m)


def kernel(node_feats, W_emb, W_mlp1, W_mlp2, W_mlp3, species_embed, W_pos,
           n_node, target_species):
    fidx = jnp.concatenate(
        [jnp.zeros((1,), jnp.int32), jnp.cumsum(n_node)[:-1].astype(jnp.int32)])
    f32 = jnp.float32
    species_logits, species_probs, coeffs = pl.pallas_call(
        _head_body,
        out_shape=[
            jax.ShapeDtypeStruct((NUM_NODES, NUM_SPECIES + 1), f32),
            jax.ShapeDtypeStruct((NUM_NODES, NUM_SPECIES + 1), f32),
            jax.ShapeDtypeStruct((NUM_GRAPHS, N_RADII * N_COEFFS), f32),
        ],
    )(node_feats, W_emb, W_mlp1, W_mlp2, W_mlp3, species_embed, W_pos,
      fidx.reshape(NUM_GRAPHS, 1),
      target_species.astype(jnp.int32).reshape(NUM_GRAPHS, 1))

    position_coeffs = coeffs.reshape(NUM_GRAPHS, N_RADII, N_COEFFS)
    rb = N_RADII * RES_BETA
    lg_flat, pb_flat = pl.pallas_call(
        _pos_body,
        grid=(NUM_GRAPHS,),
        in_specs=[
            pl.BlockSpec((1, N_RADII, N_COEFFS), lambda g: (g, 0, 0)),
            pl.BlockSpec((rb, N_RADII), lambda g: (0, 0)),
            pl.BlockSpec((rb, N_COEFFS), lambda g: (0, 0)),
            pl.BlockSpec((N_COEFFS, RES_ALPHA), lambda g: (0, 0)),
            pl.BlockSpec((N_COEFFS, RES_ALPHA), lambda g: (0, 0)),
        ],
        out_specs=[
            pl.BlockSpec((1, rb, RES_ALPHA), lambda g: (g, 0, 0)),
            pl.BlockSpec((1, rb, RES_ALPHA), lambda g: (g, 0, 0)),
        ],
        out_shape=[
            jax.ShapeDtypeStruct((NUM_GRAPHS, rb, RES_ALPHA), f32),
            jax.ShapeDtypeStruct((NUM_GRAPHS, rb, RES_ALPHA), f32),
        ],
    )(position_coeffs, jnp.asarray(_R40), jnp.asarray(_PP_TILE),
      jnp.asarray(_TRIG_HI), jnp.asarray(_TRIG_LO))

    position_logits = lg_flat.reshape(NUM_GRAPHS, N_RADII, RES_BETA, RES_ALPHA)
    position_probs = pb_flat.reshape(NUM_GRAPHS, N_RADII, RES_BETA, RES_ALPHA)
    return (species_logits, species_probs, position_coeffs, position_logits,
            position_probs, fidx)
